# baseline (speedup 1.0000x reference)
import functools

import jax
import jax.numpy as jnp
from jax import lax
from jax.experimental import pallas as pl
from jax.experimental.pallas import tpu as pltpu
from jax.experimental.pallas import tpu_sc as plsc

_N = 10000
_E = 320000
_D = 128
_DE = 16

_NC = 2
_NS = 16
_NW = _NC * _NS
_PER_W = _E // _NW
_CH = 80
_NIT = _PER_W // _CH
_NHALF = _N // _NC
_ACC_ROWS = 5120
_SROWS = 320
_RCH = 40
_EPT = _E // _NS
_SNIT = _EPT // _CH

_NBLK = 2000
_F32 = jnp.float32


def _pre_body(x_ref, w_ref, b_ref, t1_ref, t2_ref, p_ref):
    out = jnp.dot(x_ref[...], w_ref[...], preferred_element_type=_F32) + b_ref[...]
    t1_ref[...] = out[:, : 2 * _D]
    t2_ref[...] = out[:, 2 * _D : 3 * _D]
    p_ref[...] = out[:, 3 * _D :]


def _precompute(x, wcat, bcat):
    grid = (_N // _NBLK,)
    return pl.pallas_call(
        _pre_body,
        grid=grid,
        in_specs=[
            pl.BlockSpec((_NBLK, _D), lambda i: (i, 0)),
            pl.BlockSpec((_D, 4 * _D), lambda i: (0, 0)),
            pl.BlockSpec((1, 4 * _D), lambda i: (0, 0)),
        ],
        out_specs=[
            pl.BlockSpec((_NBLK, 2 * _D), lambda i: (i, 0)),
            pl.BlockSpec((_NBLK, _D), lambda i: (i, 0)),
            pl.BlockSpec((_NBLK, _D), lambda i: (i, 0)),
        ],
        out_shape=[
            jax.ShapeDtypeStruct((_N, 2 * _D), _F32),
            jax.ShapeDtypeStruct((_N, _D), _F32),
            jax.ShapeDtypeStruct((_N, _D), _F32),
        ],
    )(x, wcat, bcat)


def _gather_body(t1, t2, row, col, g1, g2, *r):
    cid = lax.axis_index("c")
    sid = lax.axis_index("s")
    base = (sid * _NC + cid) * _PER_W

    sets = []
    for k in range(3):
        b = r[4 * k: 4 * k + 4]
        s = r[12 + 4 * k: 16 + 4 * k]
        sets.append(dict(ir=b[0], ic=b[1], b1=b[2], b2=b[3],
                         sg1=s[0], sg2=s[1], sw1=s[2], sw2=s[3]))
    SA, SB, SC = sets

    def load_idx(S, i):
        off = base + i * _CH
        pltpu.sync_copy(row.at[pl.ds(off, _CH)], S["ir"])
        pltpu.sync_copy(col.at[pl.ds(off, _CH)], S["ic"])

    def start_gather(S):
        pltpu.async_copy(t1.at[S["ir"]], S["b1"], S["sg1"])
        pltpu.async_copy(t2.at[S["ic"]], S["b2"], S["sg2"])

    def wait_gather(S):
        pltpu.make_async_copy(t1.at[S["ir"]], S["b1"], S["sg1"]).wait()
        pltpu.make_async_copy(t2.at[S["ic"]], S["b2"], S["sg2"]).wait()

    def start_writeout(S, i):
        off = base + i * _CH
        pltpu.async_copy(S["b1"], g1.at[pl.ds(off, _CH)], S["sw1"])
        pltpu.async_copy(S["b2"], g2.at[pl.ds(off, _CH)], S["sw2"])

    def wait_writeout(S, i):
        off = base + i * _CH
        pltpu.make_async_copy(S["b1"], g1.at[pl.ds(off, _CH)], S["sw1"]).wait()
        pltpu.make_async_copy(S["b2"], g2.at[pl.ds(off, _CH)], S["sw2"]).wait()

    def steady(i, S0, S1, S2, first=False, do_g=True, do_idx=True):
        if not first:
            wait_writeout(S2, i - 1)
        if do_g:
            start_gather(S2)
        wait_gather(S0)
        start_writeout(S0, i)
        if do_idx:
            load_idx(S0, i + 3)

    load_idx(SA, 0)
    load_idx(SB, 1)
    load_idx(SC, 2)
    start_gather(SA)
    start_gather(SB)
    steady(0, SA, SB, SC, first=True)

    def body(k, carry):
        steady(3 * k + 1, SB, SC, SA)
        steady(3 * k + 2, SC, SA, SB)
        steady(3 * k + 3, SA, SB, SC)
        return carry

    lax.fori_loop(0, (_NIT - 5) // 3, body, 0)
    steady(_NIT - 4, SB, SC, SA)
    steady(_NIT - 3, SC, SA, SB, do_idx=False)
    steady(_NIT - 2, SA, SB, SC, do_g=False, do_idx=False)
    steady(_NIT - 1, SB, SC, SA, do_g=False, do_idx=False)
    wait_writeout(SB, _NIT - 1)


def _gather(t1, t2, row, col):
    mesh = plsc.VectorSubcoreMesh(core_axis_name="c", subcore_axis_name="s")
    fn = functools.partial(
        pl.kernel,
        out_type=[
            jax.ShapeDtypeStruct((_E, 2 * _D), _F32),
            jax.ShapeDtypeStruct((_E, _D), _F32),
        ],
        mesh=mesh,
        scratch_types=[
            pltpu.VMEM((_CH,), jnp.int32),
            pltpu.VMEM((_CH,), jnp.int32),
            pltpu.VMEM((_CH, 2 * _D), _F32),
            pltpu.VMEM((_CH, _D), _F32),
        ] * 3 + [pltpu.SemaphoreType.DMA] * 12,
    )(_gather_body)
    return fn(t1, t2, row, col)


def _edge_body(g1, g2, ea, w1e, w2, b2r, w3e, ean_ref, m_ref):
    g1v = g1[...]
    pre_h = g1v[:, :_D] + g2[...] + jnp.dot(
        ea[...], w1e[...], preferred_element_type=_F32)
    h = jnp.maximum(pre_h, 0.0)
    e_new = jnp.dot(h, w2[...], preferred_element_type=_F32) + b2r[...]
    ean_ref[...] = e_new
    m_ref[...] = jnp.maximum(
        g1v[:, _D:] + jnp.dot(e_new, w3e[...], preferred_element_type=_F32), 0.0)


def _edge_mlp(g1, g2, ea, w1e, w2, b2r, w3e):
    grid = (_E // _NBLK,)
    return pl.pallas_call(
        _edge_body,
        grid=grid,
        in_specs=[
            pl.BlockSpec((_NBLK, 2 * _D), lambda i: (i, 0)),
            pl.BlockSpec((_NBLK, _D), lambda i: (i, 0)),
            pl.BlockSpec((_NBLK, _DE), lambda i: (i, 0)),
            pl.BlockSpec((_DE, _D), lambda i: (0, 0)),
            pl.BlockSpec((_D, _DE), lambda i: (0, 0)),
            pl.BlockSpec((1, _DE), lambda i: (0, 0)),
            pl.BlockSpec((_DE, _D), lambda i: (0, 0)),
        ],
        out_specs=[
            pl.BlockSpec((_NBLK, _DE), lambda i: (i, 0)),
            pl.BlockSpec((_NBLK, _D), lambda i: (i, 0)),
        ],
        out_shape=[
            jax.ShapeDtypeStruct((_E, _DE), _F32),
            jax.ShapeDtypeStruct((_E, _D), _F32),
        ],
    )(g1, g2, ea, w1e, w2, b2r, w3e)


def _scatter_body(m, col, zm, ones, aggm, aggc,
                  ivA, i2A, mbA, ivB, i2B, mbB, onesbuf, zbuf, accm, accc,
                  silA, smlA, samA, sacA, silB, smlB, samB, sacB):
    cid = lax.axis_index("c")
    sid = lax.axis_index("s")
    lo = cid * _NHALF
    rows0 = sid * _SROWS

    pltpu.sync_copy(zm, zbuf)
    pltpu.sync_copy(ones, onesbuf)

    def zero_body(i, carry):
        r = rows0 + i * _RCH
        pltpu.sync_copy(zbuf, accm.at[pl.ds(r, _RCH)])
        pltpu.sync_copy(zbuf, accc.at[pl.ds(r, _RCH)])
        return carry

    lax.fori_loop(0, _SROWS // _RCH, zero_body, 0)
    plsc.subcore_barrier()

    base = sid * _EPT
    trash = _NHALF + sid

    SA = dict(iv=ivA, i2=i2A, mb=mbA, sil=silA, sml=smlA, sam=samA, sac=sacA)
    SB = dict(iv=ivB, i2=i2B, mb=mbB, sil=silB, sml=smlB, sam=samB, sac=sacB)

    def start_loads(S, i):
        off = base + i * _CH
        pltpu.async_copy(col.at[pl.ds(off, _CH)], S["iv"], S["sil"])
        pltpu.async_copy(m.at[pl.ds(off, _CH)], S["mb"], S["sml"])

    def wait_load_idx(S, i):
        off = base + i * _CH
        pltpu.make_async_copy(col.at[pl.ds(off, _CH)], S["iv"], S["sil"]).wait()

    def wait_load_m(S, i):
        off = base + i * _CH
        pltpu.make_async_copy(m.at[pl.ds(off, _CH)], S["mb"], S["sml"]).wait()

    def compute_idx2(S):
        for j in range(_CH // 16):
            v = S["iv"][pl.ds(j * 16, 16)]
            rel = v - lo
            ok = (rel >= 0) & (rel < _NHALF)
            S["i2"][pl.ds(j * 16, 16)] = jnp.where(ok, rel, trash)

    def start_adds(S):
        pltpu.async_copy(S["mb"], accm.at[S["i2"]], S["sam"], add=True)
        pltpu.async_copy(onesbuf, accc.at[S["i2"]], S["sac"], add=True)

    def wait_adds(S):
        pltpu.make_async_copy(S["mb"], accm.at[S["i2"]], S["sam"]).wait()
        pltpu.make_async_copy(onesbuf, accc.at[S["i2"]], S["sac"]).wait()

    def prime(S, i):
        start_loads(S, i)
        wait_load_idx(S, i)
        compute_idx2(S)
        wait_load_m(S, i)
        start_adds(S)

    def steady(i, cur, nxt, first=False, do_next=True):
        if not first:
            wait_adds(nxt)
        if do_next:
            prime(nxt, i + 1)

    prime(SA, 0)
    steady(0, SA, SB, first=True)

    def body(k, carry):
        steady(1 + 2 * k, SB, SA)
        steady(2 + 2 * k, SA, SB)
        return carry

    lax.fori_loop(0, (_SNIT - 2) // 2, body, 0)
    wait_adds(SA)
    wait_adds(SB)
    plsc.subcore_barrier()

    nch = jnp.where(sid == _NS - 1, 5, _SROWS // _RCH)

    def out_body(i, carry):
        r = rows0 + i * _RCH
        pltpu.sync_copy(accm.at[pl.ds(r, _RCH)], zbuf)
        pltpu.sync_copy(zbuf, aggm.at[pl.ds(lo + r, _RCH)])
        pltpu.sync_copy(accc.at[pl.ds(r, _RCH)], zbuf)
        pltpu.sync_copy(zbuf, aggc.at[pl.ds(lo + r, _RCH)])
        return carry

    lax.fori_loop(0, nch, out_body, 0)


def _scatter(m, col, zm, ones):
    mesh = plsc.VectorSubcoreMesh(core_axis_name="c", subcore_axis_name="s")
    fn = functools.partial(
        pl.kernel,
        out_type=[
            jax.ShapeDtypeStruct((_N, _D), _F32),
            jax.ShapeDtypeStruct((_N, _D), _F32),
        ],
        mesh=mesh,
        scratch_types=[
            pltpu.VMEM((_CH,), jnp.int32),
            pltpu.VMEM((_CH,), jnp.int32),
            pltpu.VMEM((_CH, _D), _F32),
            pltpu.VMEM((_CH,), jnp.int32),
            pltpu.VMEM((_CH,), jnp.int32),
            pltpu.VMEM((_CH, _D), _F32),
            pltpu.VMEM((_CH, _D), _F32),
            pltpu.VMEM((_RCH, _D), _F32),
            pltpu.VMEM_SHARED((_ACC_ROWS, _D), _F32),
            pltpu.VMEM_SHARED((_ACC_ROWS, _D), _F32),
        ] + [pltpu.SemaphoreType.DMA] * 8,
    )(_scatter_body)
    return fn(m, col, zm, ones)


def _final_body(p, a, c, w4a, out):
    cnt = c[...][:, :1]
    agg = a[...] / jnp.maximum(cnt, 1.0)
    out[...] = p[...] + jnp.dot(agg, w4a[...], preferred_element_type=_F32)


def _final(p, aggm, aggc, w4a):
    grid = (_N // _NBLK,)
    return pl.pallas_call(
        _final_body,
        grid=grid,
        in_specs=[
            pl.BlockSpec((_NBLK, _D), lambda i: (i, 0)),
            pl.BlockSpec((_NBLK, _D), lambda i: (i, 0)),
            pl.BlockSpec((_NBLK, _D), lambda i: (i, 0)),
            pl.BlockSpec((_D, _D), lambda i: (0, 0)),
        ],
        out_specs=pl.BlockSpec((_NBLK, _D), lambda i: (i, 0)),
        out_shape=jax.ShapeDtypeStruct((_N, _D), _F32),
    )(p, aggm, aggc, w4a)


def kernel(x, edge_index, edge_attr, W1, b1, W2, b2, W3, b3, W4, b4):
    row = edge_index[0].astype(jnp.int32)
    col = edge_index[1].astype(jnp.int32)

    W1s, W1d, W1e = W1[:_D], W1[_D:2 * _D], W1[2 * _D:]
    W3s, W3e = W3[:_D], W3[_D:]
    W4x, W4a = W4[:_D], W4[_D:]
    wcat = jnp.concatenate([W1s, W3s, W1d, W4x], axis=1)
    bcat = jnp.concatenate(
        [b1, b3, jnp.zeros_like(b1), b4])[None, :]

    t1, t2, p = _precompute(x, wcat, bcat)
    g1, g2 = _gather(t1, t2, row, col)
    ean, m = _edge_mlp(g1, g2, edge_attr, W1e, W2, b2[None, :], W3e)

    zm = jnp.zeros((_RCH, _D), _F32)
    ones = jnp.zeros((_CH, _D), _F32).at[:, 0].set(1.0)
    aggm, aggc = _scatter(m, col, zm, ones)

    x_new = _final(p, aggm, aggc, W4a)
    return (x_new, ean)


# speedup vs baseline: 1.1476x; 1.1476x over previous
import functools

import jax
import jax.numpy as jnp
from jax import lax
from jax.experimental import pallas as pl
from jax.experimental.pallas import tpu as pltpu
from jax.experimental.pallas import tpu_sc as plsc

_N = 10000
_E = 320000
_D = 128
_DE = 16

_NC = 2
_NS = 16
_NW = _NC * _NS
_PER_W = _E // _NW
_CH = 80
_NIT = _PER_W // _CH
_NHALF = _N // _NC
_ACC_ROWS = 5120
_SROWS = 320
_RCH = 40
_EPT = _E // _NS
_SNIT = _EPT // _CH

_NBLK = 2000
_F32 = jnp.float32


def _rn16(v):
    u = jax.lax.bitcast_convert_type(v, jnp.uint32)
    return (u + jnp.uint32(0x7FFF) + ((u >> 16) & jnp.uint32(1))) >> 16


def _pre_body(x_ref, w_ref, b_ref, t1_ref, t2_ref, p_ref):
    out = jnp.dot(x_ref[...], w_ref[...], preferred_element_type=_F32) + b_ref[...]
    word = _rn16(out[:, :_D]) | (_rn16(out[:, _D:2 * _D]) << 16)
    t1_ref[...] = jax.lax.bitcast_convert_type(word, jnp.int32)
    t2_ref[...] = out[:, 2 * _D : 3 * _D]
    p_ref[...] = out[:, 3 * _D :]


def _precompute(x, wcat, bcat):
    grid = (_N // _NBLK,)
    return pl.pallas_call(
        _pre_body,
        grid=grid,
        in_specs=[
            pl.BlockSpec((_NBLK, _D), lambda i: (i, 0)),
            pl.BlockSpec((_D, 4 * _D), lambda i: (0, 0)),
            pl.BlockSpec((1, 4 * _D), lambda i: (0, 0)),
        ],
        out_specs=[
            pl.BlockSpec((_NBLK, _D), lambda i: (i, 0)),
            pl.BlockSpec((_NBLK, _D), lambda i: (i, 0)),
            pl.BlockSpec((_NBLK, _D), lambda i: (i, 0)),
        ],
        out_shape=[
            jax.ShapeDtypeStruct((_N, _D), jnp.int32),
            jax.ShapeDtypeStruct((_N, _D), _F32),
            jax.ShapeDtypeStruct((_N, _D), _F32),
        ],
    )(x, wcat, bcat)


def _gather_body(t1, t2, row, col, g1, g2, *r):
    cid = lax.axis_index("c")
    sid = lax.axis_index("s")
    base = (sid * _NC + cid) * _PER_W

    sets = []
    for k in range(3):
        b = r[4 * k: 4 * k + 4]
        s = r[12 + 4 * k: 16 + 4 * k]
        sets.append(dict(ir=b[0], ic=b[1], b1=b[2], b2=b[3],
                         sg1=s[0], sg2=s[1], sw1=s[2], sw2=s[3]))
    SA, SB, SC = sets

    def load_idx(S, i):
        off = base + i * _CH
        pltpu.sync_copy(row.at[pl.ds(off, _CH)], S["ir"])
        pltpu.sync_copy(col.at[pl.ds(off, _CH)], S["ic"])

    def start_gather(S):
        pltpu.async_copy(t1.at[S["ir"]], S["b1"], S["sg1"])
        pltpu.async_copy(t2.at[S["ic"]], S["b2"], S["sg2"])

    def wait_gather(S):
        pltpu.make_async_copy(t1.at[S["ir"]], S["b1"], S["sg1"]).wait()
        pltpu.make_async_copy(t2.at[S["ic"]], S["b2"], S["sg2"]).wait()

    def start_writeout(S, i):
        off = base + i * _CH
        pltpu.async_copy(S["b1"], g1.at[pl.ds(off, _CH)], S["sw1"])
        pltpu.async_copy(S["b2"], g2.at[pl.ds(off, _CH)], S["sw2"])

    def wait_writeout(S, i):
        off = base + i * _CH
        pltpu.make_async_copy(S["b1"], g1.at[pl.ds(off, _CH)], S["sw1"]).wait()
        pltpu.make_async_copy(S["b2"], g2.at[pl.ds(off, _CH)], S["sw2"]).wait()

    def steady(i, S0, S1, S2, first=False, do_g=True, do_idx=True):
        if not first:
            wait_writeout(S2, i - 1)
        if do_g:
            start_gather(S2)
        wait_gather(S0)
        start_writeout(S0, i)
        if do_idx:
            load_idx(S0, i + 3)

    load_idx(SA, 0)
    load_idx(SB, 1)
    load_idx(SC, 2)
    start_gather(SA)
    start_gather(SB)
    steady(0, SA, SB, SC, first=True)

    def body(k, carry):
        steady(3 * k + 1, SB, SC, SA)
        steady(3 * k + 2, SC, SA, SB)
        steady(3 * k + 3, SA, SB, SC)
        return carry

    lax.fori_loop(0, (_NIT - 5) // 3, body, 0)
    steady(_NIT - 4, SB, SC, SA)
    steady(_NIT - 3, SC, SA, SB, do_idx=False)
    steady(_NIT - 2, SA, SB, SC, do_g=False, do_idx=False)
    steady(_NIT - 1, SB, SC, SA, do_g=False, do_idx=False)
    wait_writeout(SB, _NIT - 1)


def _gather(t1, t2, row, col):
    mesh = plsc.VectorSubcoreMesh(core_axis_name="c", subcore_axis_name="s")
    fn = functools.partial(
        pl.kernel,
        out_type=[
            jax.ShapeDtypeStruct((_E, _D), jnp.int32),
            jax.ShapeDtypeStruct((_E, _D), _F32),
        ],
        mesh=mesh,
        scratch_types=[
            pltpu.VMEM((_CH,), jnp.int32),
            pltpu.VMEM((_CH,), jnp.int32),
            pltpu.VMEM((_CH, _D), jnp.int32),
            pltpu.VMEM((_CH, _D), _F32),
        ] * 3 + [pltpu.SemaphoreType.DMA] * 12,
    )(_gather_body)
    return fn(t1, t2, row, col)


def _edge_body(g1, g2, ea, w1e, w2, b2r, w3e, ean_ref, m_ref):
    u = jax.lax.bitcast_convert_type(g1[...], jnp.uint32)
    a_part = jax.lax.bitcast_convert_type(u << 16, _F32)
    c_part = jax.lax.bitcast_convert_type(u & jnp.uint32(0xFFFF0000), _F32)
    pre_h = a_part + g2[...] + jnp.dot(
        ea[...], w1e[...], preferred_element_type=_F32)
    h = jnp.maximum(pre_h, 0.0)
    e_new = jnp.dot(h, w2[...], preferred_element_type=_F32) + b2r[...]
    ean_ref[...] = e_new
    m_ref[...] = jnp.maximum(
        c_part + jnp.dot(e_new, w3e[...], preferred_element_type=_F32), 0.0)


def _edge_mlp(g1, g2, ea, w1e, w2, b2r, w3e):
    grid = (_E // _NBLK,)
    return pl.pallas_call(
        _edge_body,
        grid=grid,
        in_specs=[
            pl.BlockSpec((_NBLK, _D), lambda i: (i, 0)),
            pl.BlockSpec((_NBLK, _D), lambda i: (i, 0)),
            pl.BlockSpec((_NBLK, _DE), lambda i: (i, 0)),
            pl.BlockSpec((_DE, _D), lambda i: (0, 0)),
            pl.BlockSpec((_D, _DE), lambda i: (0, 0)),
            pl.BlockSpec((1, _DE), lambda i: (0, 0)),
            pl.BlockSpec((_DE, _D), lambda i: (0, 0)),
        ],
        out_specs=[
            pl.BlockSpec((_NBLK, _DE), lambda i: (i, 0)),
            pl.BlockSpec((_NBLK, _D), lambda i: (i, 0)),
        ],
        out_shape=[
            jax.ShapeDtypeStruct((_E, _DE), _F32),
            jax.ShapeDtypeStruct((_E, _D), _F32),
        ],
    )(g1, g2, ea, w1e, w2, b2r, w3e)


def _scatter_body(m, col, zm, ones, aggm, aggc,
                  ivA, i2A, mbA, ivB, i2B, mbB, onesbuf, zbuf, accm, accc,
                  silA, smlA, samA, sacA, silB, smlB, samB, sacB):
    cid = lax.axis_index("c")
    sid = lax.axis_index("s")
    lo = cid * _NHALF
    rows0 = sid * _SROWS

    pltpu.sync_copy(zm, zbuf)
    pltpu.sync_copy(ones, onesbuf)

    def zero_body(i, carry):
        r = rows0 + i * _RCH
        pltpu.sync_copy(zbuf, accm.at[pl.ds(r, _RCH)])
        pltpu.sync_copy(zbuf, accc.at[pl.ds(r, _RCH)])
        return carry

    lax.fori_loop(0, _SROWS // _RCH, zero_body, 0)
    plsc.subcore_barrier()

    base = sid * _EPT
    trash = _NHALF + sid

    SA = dict(iv=ivA, i2=i2A, mb=mbA, sil=silA, sml=smlA, sam=samA, sac=sacA)
    SB = dict(iv=ivB, i2=i2B, mb=mbB, sil=silB, sml=smlB, sam=samB, sac=sacB)

    def start_loads(S, i):
        off = base + i * _CH
        pltpu.async_copy(col.at[pl.ds(off, _CH)], S["iv"], S["sil"])
        pltpu.async_copy(m.at[pl.ds(off, _CH)], S["mb"], S["sml"])

    def wait_load_idx(S, i):
        off = base + i * _CH
        pltpu.make_async_copy(col.at[pl.ds(off, _CH)], S["iv"], S["sil"]).wait()

    def wait_load_m(S, i):
        off = base + i * _CH
        pltpu.make_async_copy(m.at[pl.ds(off, _CH)], S["mb"], S["sml"]).wait()

    def compute_idx2(S):
        for j in range(_CH // 16):
            v = S["iv"][pl.ds(j * 16, 16)]
            rel = v - lo
            ok = (rel >= 0) & (rel < _NHALF)
            S["i2"][pl.ds(j * 16, 16)] = jnp.where(ok, rel, trash)

    def start_adds(S):
        pltpu.async_copy(S["mb"], accm.at[S["i2"]], S["sam"], add=True)
        pltpu.async_copy(onesbuf, accc.at[S["i2"]], S["sac"], add=True)

    def wait_adds(S):
        pltpu.make_async_copy(S["mb"], accm.at[S["i2"]], S["sam"]).wait()
        pltpu.make_async_copy(onesbuf, accc.at[S["i2"]], S["sac"]).wait()

    def prime(S, i):
        start_loads(S, i)
        wait_load_idx(S, i)
        compute_idx2(S)
        wait_load_m(S, i)
        start_adds(S)

    def steady(i, cur, nxt, first=False, do_next=True):
        if not first:
            wait_adds(nxt)
        if do_next:
            prime(nxt, i + 1)

    prime(SA, 0)
    steady(0, SA, SB, first=True)

    def body(k, carry):
        steady(1 + 2 * k, SB, SA)
        steady(2 + 2 * k, SA, SB)
        return carry

    lax.fori_loop(0, (_SNIT - 2) // 2, body, 0)
    wait_adds(SA)
    wait_adds(SB)
    plsc.subcore_barrier()

    nch = jnp.where(sid == _NS - 1, 5, _SROWS // _RCH)

    def out_body(i, carry):
        r = rows0 + i * _RCH
        pltpu.sync_copy(accm.at[pl.ds(r, _RCH)], zbuf)
        pltpu.sync_copy(zbuf, aggm.at[pl.ds(lo + r, _RCH)])
        pltpu.sync_copy(accc.at[pl.ds(r, _RCH)], zbuf)
        pltpu.sync_copy(zbuf, aggc.at[pl.ds(lo + r, _RCH)])
        return carry

    lax.fori_loop(0, nch, out_body, 0)


def _scatter(m, col, zm, ones):
    mesh = plsc.VectorSubcoreMesh(core_axis_name="c", subcore_axis_name="s")
    fn = functools.partial(
        pl.kernel,
        out_type=[
            jax.ShapeDtypeStruct((_N, _D), _F32),
            jax.ShapeDtypeStruct((_N, _D), _F32),
        ],
        mesh=mesh,
        scratch_types=[
            pltpu.VMEM((_CH,), jnp.int32),
            pltpu.VMEM((_CH,), jnp.int32),
            pltpu.VMEM((_CH, _D), _F32),
            pltpu.VMEM((_CH,), jnp.int32),
            pltpu.VMEM((_CH,), jnp.int32),
            pltpu.VMEM((_CH, _D), _F32),
            pltpu.VMEM((_CH, _D), _F32),
            pltpu.VMEM((_RCH, _D), _F32),
            pltpu.VMEM_SHARED((_ACC_ROWS, _D), _F32),
            pltpu.VMEM_SHARED((_ACC_ROWS, _D), _F32),
        ] + [pltpu.SemaphoreType.DMA] * 8,
    )(_scatter_body)
    return fn(m, col, zm, ones)


def _final_body(p, a, c, w4a, out):
    cnt = c[...][:, :1]
    agg = a[...] / jnp.maximum(cnt, 1.0)
    out[...] = p[...] + jnp.dot(agg, w4a[...], preferred_element_type=_F32)


def _final(p, aggm, aggc, w4a):
    grid = (_N // _NBLK,)
    return pl.pallas_call(
        _final_body,
        grid=grid,
        in_specs=[
            pl.BlockSpec((_NBLK, _D), lambda i: (i, 0)),
            pl.BlockSpec((_NBLK, _D), lambda i: (i, 0)),
            pl.BlockSpec((_NBLK, _D), lambda i: (i, 0)),
            pl.BlockSpec((_D, _D), lambda i: (0, 0)),
        ],
        out_specs=pl.BlockSpec((_NBLK, _D), lambda i: (i, 0)),
        out_shape=jax.ShapeDtypeStruct((_N, _D), _F32),
    )(p, aggm, aggc, w4a)


def kernel(x, edge_index, edge_attr, W1, b1, W2, b2, W3, b3, W4, b4):
    row = edge_index[0].astype(jnp.int32)
    col = edge_index[1].astype(jnp.int32)

    W1s, W1d, W1e = W1[:_D], W1[_D:2 * _D], W1[2 * _D:]
    W3s, W3e = W3[:_D], W3[_D:]
    W4x, W4a = W4[:_D], W4[_D:]
    wcat = jnp.concatenate([W1s, W3s, W1d, W4x], axis=1)
    bcat = jnp.concatenate(
        [b1, b3, jnp.zeros_like(b1), b4])[None, :]

    t1, t2, p = _precompute(x, wcat, bcat)
    g1, g2 = _gather(t1, t2, row, col)
    ean, m = _edge_mlp(g1, g2, edge_attr, W1e, W2, b2[None, :], W3e)

    zm = jnp.zeros((_RCH, _D), _F32)
    ones = jnp.zeros((_CH, _D), _F32).at[:, 0].set(1.0)
    aggm, aggc = _scatter(m, col, zm, ones)

    x_new = _final(p, aggm, aggc, W4a)
    return (x_new, ean)
